# baseline (speedup 1.0000x reference)
import jax
import jax.numpy as jnp
from jax import lax
from jax.experimental import pallas as pl
from jax.experimental.pallas import tpu as pltpu
from jax.experimental.pallas import tpu_sc as plsc

NEG_SLOPE = 0.2
CHUNK = 64
NW = 32


def _tc_pre_body(x_ref, w_ref, asT_ref, adT_ref, hp_ref, as_ref, ad_ref):
    h = jnp.dot(x_ref[...], w_ref[...], preferred_element_type=jnp.float32)
    hp_ref[...] = h
    as_ref[...] = jnp.dot(h, asT_ref[...], preferred_element_type=jnp.float32)
    ad_ref[...] = jnp.dot(h, adT_ref[...], preferred_element_type=jnp.float32)


def _tc_pre(xp, w, att_s, att_d):
    npad = xp.shape[0]
    return pl.pallas_call(
        _tc_pre_body,
        out_shape=(
            jax.ShapeDtypeStruct((npad, 128), jnp.float32),
            jax.ShapeDtypeStruct((npad, 1), jnp.float32),
            jax.ShapeDtypeStruct((npad, 1), jnp.float32),
        ),
    )(xp, w, att_s.reshape(128, 1), att_d.reshape(128, 1))


def _tc_mid(acc1, den1p, b1, w2, att_s2, att_d2, n):
    npad = acc1.shape[1]

    def body(acc_ref, den_ref, b1_ref, w2_ref, asT_ref, adT_ref,
             hp2_ref, as2_ref, ad2_ref, den1_ref):
        s = acc_ref[0] + acc_ref[1]
        den = jnp.sum(den_ref[...], axis=0).reshape(npad, 1) + 1e-16
        h1 = s / den + b1_ref[...]
        rid = lax.broadcasted_iota(jnp.int32, (npad, 1), 0)
        h1 = jnp.where(rid < n, h1, 0.0)
        hp2 = jnp.dot(h1, w2_ref[...], preferred_element_type=jnp.float32)
        hp2_ref[...] = hp2
        as2_ref[...] = jnp.dot(hp2, asT_ref[...],
                               preferred_element_type=jnp.float32)
        ad2_ref[...] = jnp.dot(hp2, adT_ref[...],
                               preferred_element_type=jnp.float32)
        den1_ref[...] = den

    return pl.pallas_call(
        body,
        out_shape=(
            jax.ShapeDtypeStruct((npad, 128), jnp.float32),
            jax.ShapeDtypeStruct((npad, 1), jnp.float32),
            jax.ShapeDtypeStruct((npad, 1), jnp.float32),
            jax.ShapeDtypeStruct((npad, 1), jnp.float32),
        ),
    )(acc1, den1p, b1.reshape(1, 128), w2,
      att_s2.reshape(128, 1), att_d2.reshape(128, 1))


def _tc_fin(acc2, den2p, b2, n):
    npad = acc2.shape[1]

    def body(acc_ref, den_ref, b2_ref, out_ref):
        s = acc_ref[0] + acc_ref[1]
        den = jnp.sum(den_ref[...], axis=0).reshape(npad, 1)
        h2 = s[:n, :] / (den[:n, :] + 1e-16) + b2_ref[...]
        m = jnp.max(h2, axis=1, keepdims=True)
        z = h2 - m
        out_ref[...] = z - jnp.log(jnp.sum(jnp.exp(z), axis=1, keepdims=True))

    return pl.pallas_call(
        body,
        out_shape=jax.ShapeDtypeStruct((n, 128), jnp.float32),
    )(acc2, den2p, b2.reshape(1, 128))


def _sc_layer(sd, asrc, adst, hp, emit_w, nch0=None):
    nchunks_t = sd.shape[0] // NW
    epad = sd.shape[0] * CHUNK
    npad = asrc.shape[0]
    rows_per_tile = npad // 16
    if nch0 is None:
        nch0 = nchunks_t
    nch1 = 2 * nchunks_t - nch0
    assert nch0 % 2 == 0 and nch1 % 2 == 0 and min(nch0, nch1) >= 4

    mesh = plsc.VectorSubcoreMesh(core_axis_name="c", subcore_axis_name="s")

    out_type = [
        jax.ShapeDtypeStruct((2, npad, 128), jnp.float32),
        jax.ShapeDtypeStruct((NW, npad), jnp.float32),
    ]
    if emit_w:
        out_type.append(jax.ShapeDtypeStruct((epad,), jnp.float32))
    scratch = [
        pltpu.VMEM((npad,), jnp.float32),
        pltpu.VMEM((npad,), jnp.float32),
        pltpu.VMEM((npad,), jnp.float32),
        [pltpu.VMEM((2, CHUNK), jnp.int32)] * 2,
        [pltpu.VMEM((CHUNK,), jnp.int32)] * 2,
        [pltpu.VMEM((CHUNK,), jnp.int32)] * 2,
        [pltpu.VMEM((CHUNK,), jnp.float32)] * 2,
        [pltpu.VMEM((CHUNK, 128), jnp.float32)] * 2,
        pltpu.VMEM_SHARED((npad, 128), jnp.float32),
        [pltpu.SemaphoreType.DMA] * 2,
        [pltpu.SemaphoreType.DMA] * 2,
        [pltpu.SemaphoreType.DMA] * 2,
        [pltpu.SemaphoreType.DMA] * 2,
    ]

    def body(sd_r, asrc_r, adst_r, hp_r, *rest):
        if emit_w:
            (acc_out, den_out, w_out, asrc_t, adst_t, den_l,
             sd_c, src_c, dst_c, w_c, rows, acc_sh,
             sem_i, sem_g, sem_s, sem_w) = rest
        else:
            (acc_out, den_out, asrc_t, adst_t, den_l,
             sd_c, src_c, dst_c, w_c, rows, acc_sh,
             sem_i, sem_g, sem_s, sem_w) = rest
            w_out = None

        cid = lax.axis_index("c")
        sid = lax.axis_index("s")
        wid = cid * 16 + sid
        tile_g0 = jnp.where(cid == 0, sid * nch0, 16 * nch0 + sid * nch1)
        nch = jnp.where(cid == 0, nch0, nch1)

        pltpu.sync_copy(asrc_r, asrc_t)
        pltpu.sync_copy(adst_r, adst_t)

        def dzero_step(r, carry):
            den_l[pl.ds(r * 16, 16)] = jnp.zeros((16,), jnp.float32)
            return carry

        lax.fori_loop(0, npad // 16, dzero_step, 0)

        def rzero_step(r, carry):
            for j in range(8):
                rows[0][r, pl.ds(j * 16, 16)] = jnp.zeros((16,),
                                                          jnp.float32)
            return carry

        lax.fori_loop(0, CHUNK, rzero_step, 0)

        row0 = sid * rows_per_tile

        def zero_step(r, carry):
            pltpu.sync_copy(rows[0],
                            acc_sh.at[pl.ds(row0 + r * CHUNK, CHUNK)])
            return carry

        lax.fori_loop(0, rows_per_tile // CHUNK, zero_step, 0)
        plsc.subcore_barrier()

        def drain_i(b):
            pltpu.make_async_copy(sd_r.at[0], sd_c[b], sem_i[b]).wait()

        def drain_g(b):
            pltpu.make_async_copy(hp_r.at[pl.ds(0, CHUNK)], rows[b],
                                  sem_g[b]).wait()

        def drain_s(b):
            pltpu.make_async_copy(hp_r.at[pl.ds(0, CHUNK)], rows[b],
                                  sem_s[b]).wait()

        def drain_w(b):
            pltpu.make_async_copy(w_out.at[pl.ds(0, CHUNK)], w_c[b],
                                  sem_w[b]).wait()

        def load_idx(c, b):
            pltpu.async_copy(sd_r.at[tile_g0 + c], sd_c[b], sem_i[b])

        def compute_w(c, b):
            for v in range(CHUNK // 16):
                src_c[b][pl.ds(v * 16, 16)] = sd_c[b][0, pl.ds(v * 16, 16)]
                dv = sd_c[b][1, pl.ds(v * 16, 16)]
                dst_c[b][pl.ds(v * 16, 16)] = dv
                sv = src_c[b][pl.ds(v * 16, 16)]
                a = (plsc.load_gather(asrc_t, [sv])
                     + plsc.load_gather(adst_t, [dv]))
                e = jnp.maximum(a, NEG_SLOPE * a)
                w = jnp.exp(e)
                w_c[b][pl.ds(v * 16, 16)] = w
                plsc.addupdate_scatter(den_l, [dv], w)
            if emit_w:
                pltpu.async_copy(w_c[b],
                                 w_out.at[pl.ds((tile_g0 + c) * CHUNK,
                                                CHUNK)],
                                 sem_w[b])

        def start_gather(b):
            pltpu.async_copy(hp_r.at[src_c[b]], rows[b], sem_g[b])

        def scale_scatter(b):
            @plsc.parallel_loop(0, CHUNK, 1, unroll=8)
            def _(i):
                iv = jnp.full((16,), i, jnp.int32)
                wsp = plsc.load_gather(w_c[b], [iv])
                for v in range(128 // 16):
                    rows[b][i, pl.ds(v * 16, 16)] = (
                        rows[b][i, pl.ds(v * 16, 16)] * wsp)

            pltpu.async_copy(rows[b], acc_sh.at[dst_c[b]], sem_s[b],
                             add=True)

        pltpu.sync_copy(sd_r.at[tile_g0], sd_c[0])
        compute_w(0, 0)
        load_idx(1, 1)
        start_gather(0)

        drain_i(1)
        compute_w(1, 1)
        load_idx(2, 0)
        start_gather(1)
        drain_g(0)
        scale_scatter(0)

        def pair_step(j, carry):
            c0 = 2 * j
            drain_s(0)
            drain_i(0)
            if emit_w:
                drain_w(0)
            compute_w(c0, 0)
            load_idx(c0 + 1, 1)
            start_gather(0)
            drain_g(1)
            scale_scatter(1)
            drain_s(1)
            drain_i(1)
            if emit_w:
                drain_w(1)
            compute_w(c0 + 1, 1)

            @pl.when(j < nch // 2 - 1)
            def _():
                pltpu.async_copy(sd_r.at[tile_g0 + c0 + 2], sd_c[0],
                                 sem_i[0])

            start_gather(1)
            drain_g(0)
            scale_scatter(0)
            return carry

        lax.fori_loop(1, nch // 2, pair_step, 0)

        drain_g(1)
        scale_scatter(1)
        drain_s(0)
        drain_s(1)
        if emit_w:
            drain_w(0)
            drain_w(1)

        pltpu.sync_copy(den_l, den_out.at[wid])

        plsc.subcore_barrier()
        pltpu.sync_copy(acc_sh.at[pl.ds(row0, rows_per_tile)],
                        acc_out.at[cid, pl.ds(row0, rows_per_tile)])

    k = pl.kernel(body, mesh=mesh, out_type=tuple(out_type),
                  scratch_types=scratch,
                  compiler_params=pltpu.CompilerParams(
                      needs_layout_passes=False))
    return k(sd, asrc, adst, hp)


def _sc_alpha(sd, den1, w1):
    nchunks_t = sd.shape[0] // NW
    epad = sd.shape[0] * CHUNK
    npad = den1.shape[0]
    ept = epad // NW

    mesh = plsc.VectorSubcoreMesh(core_axis_name="c", subcore_axis_name="s")

    def body(sd_r, den1_r, w1_r, alpha_out, den_t, sd_t, w1_t, alpha_t):
        cid = lax.axis_index("c")
        sid = lax.axis_index("s")
        wid = cid * 16 + sid
        tile_e0 = wid * ept
        pltpu.sync_copy(den1_r, den_t)
        pltpu.sync_copy(sd_r.at[pl.ds(wid * nchunks_t, nchunks_t)], sd_t)
        pltpu.sync_copy(w1_r.at[pl.ds(tile_e0, ept)], w1_t)

        def chunk_step(k, carry):
            for v in range(CHUNK // 16):
                dv = sd_t[k, 1, pl.ds(v * 16, 16)]
                den = plsc.load_gather(den_t, [dv])
                alpha_t[pl.ds(k * CHUNK + v * 16, 16)] = (
                    w1_t[pl.ds(k * CHUNK + v * 16, 16)] / den)
            return carry

        lax.fori_loop(0, nchunks_t, chunk_step, 0)
        pltpu.sync_copy(alpha_t, alpha_out.at[pl.ds(tile_e0, ept)])

    k = pl.kernel(body, mesh=mesh,
                  out_type=jax.ShapeDtypeStruct((epad,), jnp.float32),
                  scratch_types=[
                      pltpu.VMEM((npad,), jnp.float32),
                      pltpu.VMEM((nchunks_t, 2, CHUNK), jnp.int32),
                      pltpu.VMEM((ept,), jnp.float32),
                      pltpu.VMEM((ept,), jnp.float32),
                  ],
                  compiler_params=pltpu.CompilerParams(
                      needs_layout_passes=False))
    return k(sd, den1, w1)


def kernel(x, edge_index, W1, att_src1, att_dst1, b1,
           W2, att_src2, att_dst2, b2):
    N = x.shape[0]
    E = edge_index.shape[1]
    etot = E + N
    epad = -(-etot // (NW * CHUNK)) * (NW * CHUNK)
    npad = -(-(N + 16) // 256) * 256

    ei = edge_index.astype(jnp.int32)
    loops = jnp.arange(N, dtype=jnp.int32)
    padv = jnp.full((epad - etot,), N, dtype=jnp.int32)
    src = jnp.concatenate([ei[0], loops, padv])
    dst = jnp.concatenate([ei[1], loops, padv])
    sd = jnp.stack([src.reshape(NW, -1, CHUNK),
                    dst.reshape(NW, -1, CHUNK)], axis=2)
    sd = sd.reshape(-1, 2, CHUNK)

    xp = jnp.pad(x, ((0, npad - N), (0, 0)))

    nch0 = 210
    hp1, asrc1, adst1 = _tc_pre(xp, W1, att_src1, att_dst1)
    acc1, den1p, w1 = _sc_layer(sd, asrc1.reshape(npad),
                                adst1.reshape(npad), hp1, True, nch0)
    hp2, asrc2, adst2, den1 = _tc_mid(acc1, den1p, b1, W2,
                                      att_src2, att_dst2, N)
    alpha = _sc_alpha(sd, den1.reshape(npad), w1)
    acc2, den2p = _sc_layer(sd, asrc2.reshape(npad),
                            adst2.reshape(npad), hp2, False, nch0)
    out = _tc_fin(acc2, den2p, b2, N)
    return out, alpha[:etot].reshape(etot, 1)


# speedup vs baseline: 1.0291x; 1.0291x over previous
import jax
import jax.numpy as jnp
from jax import lax
from jax.experimental import pallas as pl
from jax.experimental.pallas import tpu as pltpu
from jax.experimental.pallas import tpu_sc as plsc

NEG_SLOPE = 0.2
CHUNK = 64
NW = 32


def _tc_pre_body(x_ref, w_ref, asT_ref, adT_ref, hp_ref, as_ref, ad_ref):
    h = jnp.dot(x_ref[...], w_ref[...], preferred_element_type=jnp.float32)
    hp_ref[...] = h
    as_ref[...] = jnp.dot(h, asT_ref[...], preferred_element_type=jnp.float32)
    ad_ref[...] = jnp.dot(h, adT_ref[...], preferred_element_type=jnp.float32)


def _tc_pre(xp, w, att_s, att_d):
    npad = xp.shape[0]
    return pl.pallas_call(
        _tc_pre_body,
        out_shape=(
            jax.ShapeDtypeStruct((npad, 128), jnp.float32),
            jax.ShapeDtypeStruct((npad, 1), jnp.float32),
            jax.ShapeDtypeStruct((npad, 1), jnp.float32),
        ),
    )(xp, w, att_s.reshape(128, 1), att_d.reshape(128, 1))


def _tc_mid(acc1, den1p, b1, w2, att_s2, att_d2, n):
    npad = acc1.shape[1]

    def body(acc_ref, den_ref, b1_ref, w2_ref, asT_ref, adT_ref,
             hp2_ref, as2_ref, ad2_ref, den1_ref):
        s = acc_ref[0] + acc_ref[1]
        den = jnp.sum(den_ref[...], axis=0).reshape(npad, 1) + 1e-16
        h1 = s / den + b1_ref[...]
        rid = lax.broadcasted_iota(jnp.int32, (npad, 1), 0)
        h1 = jnp.where(rid < n, h1, 0.0)
        hp2 = jnp.dot(h1, w2_ref[...], preferred_element_type=jnp.float32)
        hp2_ref[...] = hp2
        as2_ref[...] = jnp.dot(hp2, asT_ref[...],
                               preferred_element_type=jnp.float32)
        ad2_ref[...] = jnp.dot(hp2, adT_ref[...],
                               preferred_element_type=jnp.float32)
        den1_ref[...] = den

    return pl.pallas_call(
        body,
        out_shape=(
            jax.ShapeDtypeStruct((npad, 128), jnp.float32),
            jax.ShapeDtypeStruct((npad, 1), jnp.float32),
            jax.ShapeDtypeStruct((npad, 1), jnp.float32),
            jax.ShapeDtypeStruct((npad, 1), jnp.float32),
        ),
    )(acc1, den1p, b1.reshape(1, 128), w2,
      att_s2.reshape(128, 1), att_d2.reshape(128, 1))


def _tc_fin(acc2, den2p, b2, n):
    npad = acc2.shape[1]

    def body(acc_ref, den_ref, b2_ref, out_ref):
        s = acc_ref[0] + acc_ref[1]
        den = jnp.sum(den_ref[...], axis=0).reshape(npad, 1)
        h2 = s[:n, :] / (den[:n, :] + 1e-16) + b2_ref[...]
        m = jnp.max(h2, axis=1, keepdims=True)
        z = h2 - m
        out_ref[...] = z - jnp.log(jnp.sum(jnp.exp(z), axis=1, keepdims=True))

    return pl.pallas_call(
        body,
        out_shape=jax.ShapeDtypeStruct((n, 128), jnp.float32),
    )(acc2, den2p, b2.reshape(1, 128))


def _sc_layer(sd, asrc, adst, hp, emit_w, nch0=None):
    nchunks_t = sd.shape[0] // NW
    epad = sd.shape[0] * CHUNK
    npad = asrc.shape[0]
    rows_per_tile = npad // 16
    if nch0 is None:
        nch0 = nchunks_t
    nch1 = 2 * nchunks_t - nch0
    assert nch0 % 2 == 0 and nch1 % 2 == 0 and min(nch0, nch1) >= 4

    mesh = plsc.VectorSubcoreMesh(core_axis_name="c", subcore_axis_name="s")

    out_type = [
        jax.ShapeDtypeStruct((2, npad, 128), jnp.float32),
        jax.ShapeDtypeStruct((NW, npad), jnp.float32),
    ]
    if emit_w:
        out_type.append(jax.ShapeDtypeStruct((epad,), jnp.float32))
    scratch = [
        pltpu.VMEM((npad,), jnp.float32),
        pltpu.VMEM((npad,), jnp.float32),
        pltpu.VMEM((npad,), jnp.float32),
        [pltpu.VMEM((2, CHUNK), jnp.int32)] * 2,
        [pltpu.VMEM((CHUNK,), jnp.int32)] * 2,
        [pltpu.VMEM((CHUNK,), jnp.int32)] * 2,
        [pltpu.VMEM((CHUNK,), jnp.float32)] * 2,
        [pltpu.VMEM((CHUNK, 128), jnp.float32)] * 2,
        pltpu.VMEM_SHARED((npad, 128), jnp.float32),
        [pltpu.SemaphoreType.DMA] * 2,
        [pltpu.SemaphoreType.DMA] * 2,
        [pltpu.SemaphoreType.DMA] * 2,
        [pltpu.SemaphoreType.DMA] * 2,
    ]

    def body(sd_r, asrc_r, adst_r, hp_r, *rest):
        if emit_w:
            (acc_out, den_out, w_out, asrc_t, adst_t, den_l,
             sd_c, src_c, dst_c, w_c, rows, acc_sh,
             sem_i, sem_g, sem_s, sem_w) = rest
        else:
            (acc_out, den_out, asrc_t, adst_t, den_l,
             sd_c, src_c, dst_c, w_c, rows, acc_sh,
             sem_i, sem_g, sem_s, sem_w) = rest
            w_out = None

        cid = lax.axis_index("c")
        sid = lax.axis_index("s")
        wid = cid * 16 + sid
        tile_g0 = jnp.where(cid == 0, sid * nch0, 16 * nch0 + sid * nch1)
        nch = jnp.where(cid == 0, nch0, nch1)

        pltpu.sync_copy(asrc_r, asrc_t)
        pltpu.sync_copy(adst_r, adst_t)

        def dzero_step(r, carry):
            den_l[pl.ds(r * 16, 16)] = jnp.zeros((16,), jnp.float32)
            return carry

        lax.fori_loop(0, npad // 16, dzero_step, 0)

        def rzero_step(r, carry):
            for j in range(8):
                rows[0][r, pl.ds(j * 16, 16)] = jnp.zeros((16,),
                                                          jnp.float32)
            return carry

        lax.fori_loop(0, CHUNK, rzero_step, 0)

        row0 = sid * rows_per_tile

        def zero_step(r, carry):
            pltpu.sync_copy(rows[0],
                            acc_sh.at[pl.ds(row0 + r * CHUNK, CHUNK)])
            return carry

        lax.fori_loop(0, rows_per_tile // CHUNK, zero_step, 0)
        plsc.subcore_barrier()

        def drain_i(b):
            pltpu.make_async_copy(sd_r.at[0], sd_c[b], sem_i[b]).wait()

        def drain_g(b):
            pltpu.make_async_copy(hp_r.at[pl.ds(0, CHUNK)], rows[b],
                                  sem_g[b]).wait()

        def drain_s(b):
            pltpu.make_async_copy(hp_r.at[pl.ds(0, CHUNK)], rows[b],
                                  sem_s[b]).wait()

        def drain_w(b):
            pltpu.make_async_copy(w_out.at[pl.ds(0, CHUNK)], w_c[b],
                                  sem_w[b]).wait()

        def load_idx(c, b):
            pltpu.async_copy(sd_r.at[tile_g0 + c], sd_c[b], sem_i[b])

        def compute_w(c, b):
            for v in range(CHUNK // 16):
                src_c[b][pl.ds(v * 16, 16)] = sd_c[b][0, pl.ds(v * 16, 16)]
                dv = sd_c[b][1, pl.ds(v * 16, 16)]
                dst_c[b][pl.ds(v * 16, 16)] = dv
                sv = src_c[b][pl.ds(v * 16, 16)]
                a = (plsc.load_gather(asrc_t, [sv])
                     + plsc.load_gather(adst_t, [dv]))
                e = jnp.maximum(a, NEG_SLOPE * a)
                w = jnp.exp(e)
                w_c[b][pl.ds(v * 16, 16)] = w
                plsc.addupdate_scatter(den_l, [dv], w)
            if emit_w:
                pltpu.async_copy(w_c[b],
                                 w_out.at[pl.ds((tile_g0 + c) * CHUNK,
                                                CHUNK)],
                                 sem_w[b])

        def start_gather(b):
            pltpu.async_copy(hp_r.at[src_c[b]], rows[b], sem_g[b])

        def scale_scatter(b):
            @plsc.parallel_loop(0, CHUNK, 1, unroll=8)
            def _(i):
                iv = jnp.full((16,), i, jnp.int32)
                wsp = plsc.load_gather(w_c[b], [iv])
                for v in range(128 // 16):
                    rows[b][i, pl.ds(v * 16, 16)] = (
                        rows[b][i, pl.ds(v * 16, 16)] * wsp)

            pltpu.async_copy(rows[b], acc_sh.at[dst_c[b]], sem_s[b],
                             add=True)

        pltpu.sync_copy(sd_r.at[tile_g0], sd_c[0])
        compute_w(0, 0)
        load_idx(1, 1)
        start_gather(0)

        drain_i(1)
        compute_w(1, 1)
        load_idx(2, 0)
        start_gather(1)
        drain_g(0)
        scale_scatter(0)

        def pair_step(j, carry):
            c0 = 2 * j
            drain_s(0)
            drain_i(0)
            if emit_w:
                drain_w(0)
            compute_w(c0, 0)
            load_idx(c0 + 1, 1)
            start_gather(0)
            drain_g(1)
            scale_scatter(1)
            drain_s(1)
            drain_i(1)
            if emit_w:
                drain_w(1)
            compute_w(c0 + 1, 1)

            @pl.when(j < nch // 2 - 1)
            def _():
                pltpu.async_copy(sd_r.at[tile_g0 + c0 + 2], sd_c[0],
                                 sem_i[0])

            start_gather(1)
            drain_g(0)
            scale_scatter(0)
            return carry

        lax.fori_loop(1, nch // 2, pair_step, 0)

        drain_g(1)
        scale_scatter(1)
        drain_s(0)
        drain_s(1)
        if emit_w:
            drain_w(0)
            drain_w(1)

        pltpu.sync_copy(den_l, den_out.at[wid])

        plsc.subcore_barrier()
        pltpu.sync_copy(acc_sh.at[pl.ds(row0, rows_per_tile)],
                        acc_out.at[cid, pl.ds(row0, rows_per_tile)])

    k = pl.kernel(body, mesh=mesh, out_type=tuple(out_type),
                  scratch_types=scratch,
                  compiler_params=pltpu.CompilerParams(
                      needs_layout_passes=False))
    return k(sd, asrc, adst, hp)


def _sc_alpha(sd, den1, w1):
    nchunks_t = sd.shape[0] // NW
    epad = sd.shape[0] * CHUNK
    npad = den1.shape[0]
    ept = epad // NW

    mesh = plsc.VectorSubcoreMesh(core_axis_name="c", subcore_axis_name="s")

    def body(sd_r, den1_r, w1_r, alpha_out, den_t, sd_t, w1_t, alpha_t):
        cid = lax.axis_index("c")
        sid = lax.axis_index("s")
        wid = cid * 16 + sid
        tile_e0 = wid * ept
        pltpu.sync_copy(den1_r, den_t)
        pltpu.sync_copy(sd_r.at[pl.ds(wid * nchunks_t, nchunks_t)], sd_t)
        pltpu.sync_copy(w1_r.at[pl.ds(tile_e0, ept)], w1_t)

        def chunk_step(k, carry):
            for v in range(CHUNK // 16):
                dv = sd_t[k, 1, pl.ds(v * 16, 16)]
                den = plsc.load_gather(den_t, [dv])
                alpha_t[pl.ds(k * CHUNK + v * 16, 16)] = (
                    w1_t[pl.ds(k * CHUNK + v * 16, 16)] / den)
            return carry

        lax.fori_loop(0, nchunks_t, chunk_step, 0)
        pltpu.sync_copy(alpha_t, alpha_out.at[pl.ds(tile_e0, ept)])

    k = pl.kernel(body, mesh=mesh,
                  out_type=jax.ShapeDtypeStruct((epad,), jnp.float32),
                  scratch_types=[
                      pltpu.VMEM((npad,), jnp.float32),
                      pltpu.VMEM((nchunks_t, 2, CHUNK), jnp.int32),
                      pltpu.VMEM((ept,), jnp.float32),
                      pltpu.VMEM((ept,), jnp.float32),
                  ],
                  compiler_params=pltpu.CompilerParams(
                      needs_layout_passes=False))
    return k(sd, den1, w1)


def kernel(x, edge_index, W1, att_src1, att_dst1, b1,
           W2, att_src2, att_dst2, b2):
    N = x.shape[0]
    E = edge_index.shape[1]
    etot = E + N
    epad = -(-etot // (NW * CHUNK)) * (NW * CHUNK)
    npad = -(-(N + 16) // 256) * 256

    ei = edge_index.astype(jnp.int32)
    loops = jnp.arange(N, dtype=jnp.int32)
    padv = jnp.full((epad - etot,), N, dtype=jnp.int32)
    src = jnp.concatenate([ei[0], loops, padv])
    dst = jnp.concatenate([ei[1], loops, padv])
    sd = jnp.stack([src.reshape(NW, -1, CHUNK),
                    dst.reshape(NW, -1, CHUNK)], axis=2)
    sd = sd.reshape(-1, 2, CHUNK)

    xp = jnp.pad(x, ((0, npad - N), (0, 0)))

    nch0 = 196
    hp1, asrc1, adst1 = _tc_pre(xp, W1, att_src1, att_dst1)
    acc1, den1p, w1 = _sc_layer(sd, asrc1.reshape(npad),
                                adst1.reshape(npad), hp1, True, nch0)
    hp2, asrc2, adst2, den1 = _tc_mid(acc1, den1p, b1, W2,
                                      att_src2, att_dst2, N)
    alpha = _sc_alpha(sd, den1.reshape(npad), w1)
    acc2, den2p = _sc_layer(sd, asrc2.reshape(npad),
                            adst2.reshape(npad), hp2, False, nch0)
    out = _tc_fin(acc2, den2p, b2, N)
    return out, alpha[:etot].reshape(etot, 1)
